# Initial kernel scaffold; baseline (speedup 1.0000x reference)
import jax
import jax.numpy as jnp
from jax.experimental import pallas as pl


def kernel(points, centroids):
    raise NotImplementedError("write your pallas kernel here")


# speedup vs baseline: 22.2670x; 22.2670x over previous
import functools

import jax
import jax.numpy as jnp
from jax.experimental import pallas as pl


def _assign_kernel(p_ref, ct_ref, o_ref, *, k, precision):
    ct = ct_ref[...]
    cnorm = jnp.sum(ct * ct, axis=0)
    scores = cnorm[None, :] - 2.0 * jax.lax.dot_general(
        p_ref[...], ct,
        dimension_numbers=(((1,), (0,)), ((), ())),
        preferred_element_type=jnp.float32,
        precision=precision,
    )
    m = jnp.min(scores, axis=1, keepdims=True)
    idx = jax.lax.broadcasted_iota(jnp.int32, scores.shape, 1)
    o_ref[...] = jnp.min(jnp.where(scores == m, idx, k), axis=1)


def kernel(points, centroids):
    b, d = points.shape
    k = centroids.shape[0]
    blk = 128
    body = functools.partial(_assign_kernel, k=k,
                             precision=jax.lax.Precision.HIGHEST)
    return pl.pallas_call(
        body,
        grid=(b // blk,),
        in_specs=[
            pl.BlockSpec((blk, d), lambda i: (i, 0)),
            pl.BlockSpec((d, k), lambda i: (0, 0)),
        ],
        out_specs=pl.BlockSpec((blk,), lambda i: (i,)),
        out_shape=jax.ShapeDtypeStruct((b,), jnp.int32),
    )(points, centroids.T)


# speedup vs baseline: 27.6672x; 1.2425x over previous
import functools

import jax
import jax.numpy as jnp
from jax.experimental import pallas as pl
from jax.experimental.pallas import tpu as pltpu


def _assign_kernel(p_ref, c_ref, o_ref, ct_ref, cn_ref, *, k, precision):
    @pl.when(pl.program_id(0) == 0)
    def _prep():
        ct = c_ref[...].T
        ct_ref[...] = ct
        cn_ref[...] = jnp.sum(ct * ct, axis=0, keepdims=True)

    scores = cn_ref[...] - 2.0 * jax.lax.dot_general(
        p_ref[...], ct_ref[...],
        dimension_numbers=(((1,), (0,)), ((), ())),
        preferred_element_type=jnp.float32,
        precision=precision,
    )
    m = jnp.min(scores, axis=1, keepdims=True)
    idx = jax.lax.broadcasted_iota(jnp.int32, scores.shape, 1)
    o_ref[...] = jnp.min(jnp.where(scores == m, idx, k), axis=1)


def kernel(points, centroids):
    b, d = points.shape
    k = centroids.shape[0]
    blk = 128
    body = functools.partial(_assign_kernel, k=k,
                             precision=jax.lax.Precision.HIGHEST)
    return pl.pallas_call(
        body,
        grid=(b // blk,),
        in_specs=[
            pl.BlockSpec((blk, d), lambda i: (i, 0)),
            pl.BlockSpec((k, d), lambda i: (0, 0)),
        ],
        out_specs=pl.BlockSpec((blk,), lambda i: (i,)),
        out_shape=jax.ShapeDtypeStruct((b,), jnp.int32),
        scratch_shapes=[
            pltpu.VMEM((d, k), jnp.float32),
            pltpu.VMEM((1, k), jnp.float32),
        ],
    )(points, centroids)


# speedup vs baseline: 35.5868x; 1.2862x over previous
import functools

import jax
import jax.numpy as jnp
from jax.experimental import pallas as pl
from jax.experimental.pallas import tpu as pltpu


def _assign_kernel(p_ref, c_ref, o_ref, ct_ref, cn_ref, *, k, precision):
    @pl.when(pl.program_id(0) == 0)
    def _prep():
        ct = c_ref[...].T
        ct_ref[...] = ct
        cn_ref[...] = jnp.sum(ct * ct, axis=0, keepdims=True)

    scores = cn_ref[...] - 2.0 * jax.lax.dot_general(
        p_ref[...], ct_ref[...],
        dimension_numbers=(((1,), (0,)), ((), ())),
        preferred_element_type=jnp.float32,
        precision=precision,
    )
    m = jnp.min(scores, axis=1, keepdims=True)
    idx = jax.lax.broadcasted_iota(jnp.int32, scores.shape, 1)
    o_ref[...] = jnp.min(jnp.where(scores == m, idx, k), axis=1)


def kernel(points, centroids):
    b, d = points.shape
    k = centroids.shape[0]
    blk = 256
    body = functools.partial(_assign_kernel, k=k,
                             precision=jax.lax.Precision.HIGHEST)
    return pl.pallas_call(
        body,
        grid=(b // blk,),
        in_specs=[
            pl.BlockSpec((blk, d), lambda i: (i, 0)),
            pl.BlockSpec((k, d), lambda i: (0, 0)),
        ],
        out_specs=pl.BlockSpec((blk,), lambda i: (i,)),
        out_shape=jax.ShapeDtypeStruct((b,), jnp.int32),
        scratch_shapes=[
            pltpu.VMEM((d, k), jnp.float32),
            pltpu.VMEM((1, k), jnp.float32),
        ],
    )(points, centroids)


# speedup vs baseline: 36.8287x; 1.0349x over previous
import functools

import jax
import jax.numpy as jnp
from jax.experimental import pallas as pl
from jax.experimental.pallas import tpu as pltpu


def _assign_kernel(p_ref, c_ref, o_ref, ct_ref, cn_ref, *, k, precision):
    @pl.when(pl.program_id(0) == 0)
    def _prep():
        ct = c_ref[...].T
        ct_ref[...] = ct
        cn_ref[...] = jnp.sum(ct * ct, axis=0, keepdims=True)

    scores = cn_ref[...] - 2.0 * jax.lax.dot_general(
        p_ref[...], ct_ref[...],
        dimension_numbers=(((1,), (0,)), ((), ())),
        preferred_element_type=jnp.float32,
        precision=precision,
    )
    m = jnp.min(scores, axis=1, keepdims=True)
    idx = jax.lax.broadcasted_iota(jnp.int32, scores.shape, 1)
    o_ref[...] = jnp.min(jnp.where(scores == m, idx, k), axis=1)


def kernel(points, centroids):
    b, d = points.shape
    k = centroids.shape[0]
    blk = 512
    body = functools.partial(_assign_kernel, k=k,
                             precision=jax.lax.Precision.HIGHEST)
    return pl.pallas_call(
        body,
        grid=(b // blk,),
        in_specs=[
            pl.BlockSpec((blk, d), lambda i: (i, 0)),
            pl.BlockSpec((k, d), lambda i: (0, 0)),
        ],
        out_specs=pl.BlockSpec((blk,), lambda i: (i,)),
        out_shape=jax.ShapeDtypeStruct((b,), jnp.int32),
        scratch_shapes=[
            pltpu.VMEM((d, k), jnp.float32),
            pltpu.VMEM((1, k), jnp.float32),
        ],
    )(points, centroids)


# speedup vs baseline: 38.1848x; 1.0368x over previous
import functools

import jax
import jax.numpy as jnp
from jax.experimental import pallas as pl
from jax.experimental.pallas import tpu as pltpu


def _assign_kernel(p_ref, c_ref, o_ref, ct_ref, cn_ref, *, k, precision):
    @pl.when(pl.program_id(0) == 0)
    def _prep():
        ct = c_ref[...].T
        ct_ref[...] = ct
        cn_ref[...] = jnp.sum(ct * ct, axis=0, keepdims=True)

    scores = cn_ref[...] - 2.0 * jax.lax.dot_general(
        p_ref[...], ct_ref[...],
        dimension_numbers=(((1,), (0,)), ((), ())),
        preferred_element_type=jnp.float32,
        precision=precision,
    )
    m = jnp.min(scores, axis=1, keepdims=True)
    idx = jax.lax.broadcasted_iota(jnp.int32, scores.shape, 1)
    o_ref[...] = jnp.min(jnp.where(scores == m, idx, k), axis=1)


def kernel(points, centroids):
    b, d = points.shape
    k = centroids.shape[0]
    blk = 1024
    body = functools.partial(_assign_kernel, k=k,
                             precision=jax.lax.Precision.HIGHEST)
    return pl.pallas_call(
        body,
        grid=(b // blk,),
        in_specs=[
            pl.BlockSpec((blk, d), lambda i: (i, 0)),
            pl.BlockSpec((k, d), lambda i: (0, 0)),
        ],
        out_specs=pl.BlockSpec((blk,), lambda i: (i,)),
        out_shape=jax.ShapeDtypeStruct((b,), jnp.int32),
        scratch_shapes=[
            pltpu.VMEM((d, k), jnp.float32),
            pltpu.VMEM((1, k), jnp.float32),
        ],
    )(points, centroids)


# speedup vs baseline: 39.6245x; 1.0377x over previous
import functools

import jax
import jax.numpy as jnp
from jax.experimental import pallas as pl
from jax.experimental.pallas import tpu as pltpu


def _assign_kernel(p_ref, c_ref, o_ref, pt_ref, m_ref, i_ref, *, kc, nsteps):
    step = pl.program_id(0)

    @pl.when(step == 0)
    def _prep():
        pt_ref[...] = p_ref[...].T

    c = c_ref[...]
    cnorm = jnp.sum(c * c, axis=1, keepdims=True)
    scores = cnorm - 2.0 * jax.lax.dot_general(
        c, pt_ref[...],
        dimension_numbers=(((1,), (0,)), ((), ())),
        preferred_element_type=jnp.float32,
        precision=jax.lax.Precision.HIGHEST,
    )
    m = jnp.min(scores, axis=0, keepdims=True)
    iota = jax.lax.broadcasted_iota(jnp.int32, scores.shape, 0)
    idx = jnp.min(jnp.where(scores == m, iota, kc), axis=0,
                  keepdims=True) + step * kc

    @pl.when(step == 0)
    def _init():
        m_ref[...] = m
        i_ref[...] = idx

    @pl.when(step > 0)
    def _merge():
        better = m < m_ref[...]
        i_ref[...] = jnp.where(better, idx, i_ref[...])
        m_ref[...] = jnp.minimum(m, m_ref[...])

    @pl.when(step == nsteps - 1)
    def _out():
        o_ref[...] = i_ref[...]


def kernel(points, centroids):
    b, d = points.shape
    k = centroids.shape[0]
    kc = 256
    nsteps = k // kc
    body = functools.partial(_assign_kernel, kc=kc, nsteps=nsteps)
    out = pl.pallas_call(
        body,
        grid=(nsteps,),
        in_specs=[
            pl.BlockSpec((b, d), lambda i: (0, 0)),
            pl.BlockSpec((kc, d), lambda i: (i, 0)),
        ],
        out_specs=pl.BlockSpec((1, b), lambda i: (0, 0)),
        out_shape=jax.ShapeDtypeStruct((1, b), jnp.int32),
        scratch_shapes=[
            pltpu.VMEM((d, b), jnp.float32),
            pltpu.VMEM((1, b), jnp.float32),
            pltpu.VMEM((1, b), jnp.int32),
        ],
    )(points, centroids)
    return out.reshape(b)
